# baseline (speedup 1.0000x reference)
import jax
import jax.numpy as jnp
from jax.experimental import pallas as pl
from jax.experimental.pallas import tpu as pltpu

_PB = 4


def _body(t_ref, a_ref, c_ref, x_ref, eps_ref, nz_ref, x0_ref, np_ref, tg_ref):
    g = pl.program_id(0)
    np_ref[...] = eps_ref[...]
    tg_ref[...] = nz_ref[...]
    for j in range(_PB):
        ti = t_ref[g * _PB + j]
        x0_ref[j] = a_ref[ti] * x_ref[j] - c_ref[ti] * eps_ref[j]


def kernel(model_preds, x_t, x_0, noise, t,
           sqrt_recip_alphas_cumprod, sqrt_recipm1_alphas_cumprod):
    B, C, H, W = x_t.shape
    blk = pl.BlockSpec((_PB, C, H, W), lambda g, *_: (g, 0, 0, 0))
    grid_spec = pltpu.PrefetchScalarGridSpec(
        num_scalar_prefetch=3,
        grid=(B // _PB,),
        in_specs=[blk, blk, blk],
        out_specs=[blk, blk, blk],
    )
    out = jax.ShapeDtypeStruct(x_t.shape, x_t.dtype)
    x0p, np_, tg = pl.pallas_call(
        _body,
        grid_spec=grid_spec,
        out_shape=[out, out, out],
    )(t, sqrt_recip_alphas_cumprod, sqrt_recipm1_alphas_cumprod,
      x_t, model_preds, noise)
    return (np_, x0p, tg)


# speedup vs baseline: 1.2623x; 1.2623x over previous
import jax
import jax.numpy as jnp
from jax.experimental import pallas as pl
from jax.experimental.pallas import tpu as pltpu

_PB = 16


def _body(t_ref, a_ref, c_ref, x_ref, eps_ref, nz_ref, x0_ref, np_ref, tg_ref):
    g = pl.program_id(0)
    np_ref[...] = eps_ref[...]
    tg_ref[...] = nz_ref[...]
    for j in range(_PB):
        ti = t_ref[g * _PB + j]
        x0_ref[j] = a_ref[ti] * x_ref[j] - c_ref[ti] * eps_ref[j]


def kernel(model_preds, x_t, x_0, noise, t,
           sqrt_recip_alphas_cumprod, sqrt_recipm1_alphas_cumprod):
    B, C, H, W = x_t.shape
    blk = pl.BlockSpec((_PB, C, H, W), lambda g, *_: (g, 0, 0, 0))
    grid_spec = pltpu.PrefetchScalarGridSpec(
        num_scalar_prefetch=3,
        grid=(B // _PB,),
        in_specs=[blk, blk, blk],
        out_specs=[blk, blk, blk],
    )
    out = jax.ShapeDtypeStruct(x_t.shape, x_t.dtype)
    x0p, np_, tg = pl.pallas_call(
        _body,
        grid_spec=grid_spec,
        out_shape=[out, out, out],
    )(t, sqrt_recip_alphas_cumprod, sqrt_recipm1_alphas_cumprod,
      x_t, model_preds, noise)
    return (np_, x0p, tg)
